# baseline (speedup 1.0000x reference)
import jax
import jax.numpy as jnp
from jax.experimental import pallas as pl
from jax.experimental.pallas import tpu as pltpu

BN_EPS = 1e-5


def _h1_kernel(x_ref, w1_ref, h1_ref, stats_ref):
    xb = x_ref[...].astype(jnp.bfloat16)
    w1b = w1_ref[...].astype(jnp.bfloat16)
    h1 = jnp.dot(xb, w1b, preferred_element_type=jnp.float32)
    h1_ref[...] = h1
    stats_ref[...] = jnp.stack([jnp.sum(h1, axis=0),
                                jnp.sum(h1 * h1, axis=0)])[None]


def _finalize_kernel(h1_ref, stats_ref, bn1_ref, w2_ref, bn2_ref, w3a_ref,
                     onehot_ref, w3b_ref, b3_ref, out_ref):
    B = h1_ref.shape[0]
    inv_b = 1.0 / B

    totals = jnp.sum(stats_ref[...], axis=0)
    mean1 = totals[0:1, :] * inv_b
    var1 = totals[1:2, :] * inv_b - mean1 * mean1
    scale1 = bn1_ref[0:1, :] * jax.lax.rsqrt(var1 + BN_EPS)
    shift1 = bn1_ref[1:2, :] - mean1 * scale1
    h1n = jnp.maximum(h1_ref[...] * scale1 + shift1, 0.0)

    h2 = jnp.dot(h1n.astype(jnp.bfloat16), w2_ref[...].astype(jnp.bfloat16),
                 preferred_element_type=jnp.float32)
    mean2 = jnp.mean(h2, axis=0, keepdims=True)
    var2 = jnp.mean(h2 * h2, axis=0, keepdims=True) - mean2 * mean2
    scale2 = bn2_ref[0:1, :] * jax.lax.rsqrt(var2 + BN_EPS)
    shift2 = bn2_ref[1:2, :] - mean2 * scale2
    h2n = jnp.maximum(h2 * scale2 + shift2, 0.0)

    out_dim = out_ref.shape[1]
    logits = (jnp.dot(h2n.astype(jnp.bfloat16),
                      w3a_ref[...].astype(jnp.bfloat16),
                      preferred_element_type=jnp.float32)
              + jnp.dot(onehot_ref[...].astype(jnp.bfloat16),
                        w3b_ref[...].astype(jnp.bfloat16),
                        preferred_element_type=jnp.float32)
              + b3_ref[...])[:, :out_dim]
    out_ref[...] = jax.nn.sigmoid(logits)


def kernel(latent_vec, class_onehot, w1, bn1, w2, bn2, w3a_pad, w3b_pad,
           b3_pad, output_dim=3):
    B, latent_dim = latent_vec.shape
    H0 = w1.shape[1]
    H1 = w2.shape[1]
    OUTP = w3a_pad.shape[1]

    blk = 1024 if B % 1024 == 0 else B
    nblk = B // blk

    h1_flops = 2 * B * latent_dim * H0
    h1_bytes = B * latent_dim * 4 + latent_dim * H0 * 4 + B * H0 * 4
    h1, stats = pl.pallas_call(
        _h1_kernel,
        out_shape=(jax.ShapeDtypeStruct((B, H0), jnp.float32),
                   jax.ShapeDtypeStruct((nblk, 2, H0), jnp.float32)),
        grid=(nblk,),
        in_specs=[pl.BlockSpec((blk, latent_dim), lambda i: (i, 0)),
                  pl.BlockSpec((latent_dim, H0), lambda i: (0, 0))],
        out_specs=(pl.BlockSpec((blk, H0), lambda i: (i, 0)),
                   pl.BlockSpec((1, 2, H0), lambda i: (i, 0, 0))),
        compiler_params=pltpu.CompilerParams(
            dimension_semantics=("parallel",)),
        cost_estimate=pl.CostEstimate(flops=h1_flops, transcendentals=0,
                                      bytes_accessed=h1_bytes),
    )(latent_vec, w1)

    fin_flops = 2 * B * (H0 * H1 + H1 * OUTP + 16 * OUTP) + 12 * B * (H0 + H1)
    fin_bytes = (B * H0 * 4 + B * H1 * 4 + B * 16 * 4 + H0 * H1 * 4
                 + H1 * OUTP * 4 + B * output_dim * 4)
    args = (h1, stats, bn1, w2, bn2, w3a_pad, class_onehot, w3b_pad, b3_pad)
    in_specs = [pl.BlockSpec(a.shape, lambda *_, n=a.ndim: (0,) * n)
                for a in args]
    out = pl.pallas_call(
        _finalize_kernel,
        out_shape=jax.ShapeDtypeStruct((B, output_dim), jnp.float32),
        grid=(1,),
        in_specs=in_specs,
        out_specs=pl.BlockSpec((B, output_dim), lambda *_: (0, 0)),
        compiler_params=pltpu.CompilerParams(
            dimension_semantics=("arbitrary",),
            vmem_limit_bytes=64 * 1024 * 1024),
        cost_estimate=pl.CostEstimate(flops=fin_flops,
                                      transcendentals=B * output_dim + H0 + H1,
                                      bytes_accessed=fin_bytes),
    )(*args)
    return out


# speedup vs baseline: 1.2569x; 1.2569x over previous
import jax
import jax.numpy as jnp
from jax.experimental import pallas as pl
from jax.experimental.pallas import tpu as pltpu

BN_EPS = 1e-5


def _fused_kernel(x_ref, onehot_ref, w1_ref, bn1_ref, w2_ref, bn2_ref,
                  w3a_ref, w3b_ref, b3_ref, out_ref,
                  h1_ref, h2_ref, s1_ref, s2_ref):
    step = pl.program_id(0)
    nblk = pl.num_programs(0) // 3
    blk = x_ref.shape[0]
    b_total = h1_ref.shape[0]
    inv_b = 1.0 / b_total

    @pl.when(step < nblk)
    def _phase_a():
        xb = x_ref[...].astype(jnp.bfloat16)
        h1 = jnp.dot(xb, w1_ref[...].astype(jnp.bfloat16),
                     preferred_element_type=jnp.float32)
        h1_ref[pl.ds(step * blk, blk), :] = h1

        @pl.when(step == 0)
        def _():
            s1_ref[...] = jnp.zeros_like(s1_ref)

        s1_ref[...] += jnp.stack([jnp.sum(h1, axis=0),
                                  jnp.sum(h1 * h1, axis=0)])

    @pl.when((step >= nblk) & (step < 2 * nblk))
    def _phase_b():
        i = step - nblk
        totals = s1_ref[...]
        mean = totals[0:1, :] * inv_b
        var = totals[1:2, :] * inv_b - mean * mean
        scale = bn1_ref[0:1, :] * jax.lax.rsqrt(var + BN_EPS)
        shift = bn1_ref[1:2, :] - mean * scale
        h1 = h1_ref[pl.ds(i * blk, blk), :]
        h1n = jnp.maximum(h1 * scale + shift, 0.0)
        h2 = jnp.dot(h1n.astype(jnp.bfloat16),
                     w2_ref[...].astype(jnp.bfloat16),
                     preferred_element_type=jnp.float32)
        h2_ref[pl.ds(i * blk, blk), :] = h2

        @pl.when(i == 0)
        def _():
            s2_ref[...] = jnp.zeros_like(s2_ref)

        s2_ref[...] += jnp.stack([jnp.sum(h2, axis=0),
                                  jnp.sum(h2 * h2, axis=0)])

    @pl.when(step >= 2 * nblk)
    def _phase_c():
        i = step - 2 * nblk
        totals = s2_ref[...]
        mean = totals[0:1, :] * inv_b
        var = totals[1:2, :] * inv_b - mean * mean
        scale = bn2_ref[0:1, :] * jax.lax.rsqrt(var + BN_EPS)
        shift = bn2_ref[1:2, :] - mean * scale
        h2 = h2_ref[pl.ds(i * blk, blk), :]
        h2n = jnp.maximum(h2 * scale + shift, 0.0)
        oh = onehot_ref[pl.ds(i * blk, blk), :]
        out_dim = out_ref.shape[1]
        logits = (jnp.dot(h2n.astype(jnp.bfloat16),
                          w3a_ref[...].astype(jnp.bfloat16),
                          preferred_element_type=jnp.float32)
                  + jnp.dot(oh.astype(jnp.bfloat16),
                            w3b_ref[...].astype(jnp.bfloat16),
                            preferred_element_type=jnp.float32)
                  + b3_ref[...])[:, :out_dim]
        out_ref[...] = jax.nn.sigmoid(logits)


def kernel(latent_vec, class_onehot, w1, bn1, w2, bn2, w3a_pad, w3b_pad,
           b3_pad, output_dim=3):
    B, latent_dim = latent_vec.shape
    H0 = w1.shape[1]
    H1 = w2.shape[1]
    OUTP = w3a_pad.shape[1]
    C = class_onehot.shape[1]

    blk = 1024 if B % 1024 == 0 else B
    nblk = B // blk
    nsteps = 3 * nblk
    last = nblk - 1

    flops = (2 * B * (latent_dim * H0 + H0 * H1 + H1 * OUTP + C * OUTP)
             + 12 * B * (H0 + H1))
    bytes_accessed = (B * latent_dim * 4 + B * C * 4 + latent_dim * H0 * 4
                      + H0 * H1 * 4 + (H1 + C) * OUTP * 4
                      + B * output_dim * 4)

    grid_spec = pltpu.PrefetchScalarGridSpec(
        num_scalar_prefetch=0,
        grid=(nsteps,),
        in_specs=[
            pl.BlockSpec((blk, latent_dim),
                         lambda s: (jnp.minimum(s, last), 0)),
            pl.BlockSpec((B, C), lambda s: (0, 0)),
            pl.BlockSpec((latent_dim, H0), lambda s: (0, 0)),
            pl.BlockSpec((2, H0), lambda s: (0, 0)),
            pl.BlockSpec((H0, H1), lambda s: (0, 0)),
            pl.BlockSpec((2, H1), lambda s: (0, 0)),
            pl.BlockSpec((H1, OUTP), lambda s: (0, 0)),
            pl.BlockSpec((C, OUTP), lambda s: (0, 0)),
            pl.BlockSpec((1, OUTP), lambda s: (0, 0)),
        ],
        out_specs=pl.BlockSpec(
            (blk, output_dim),
            lambda s: (jnp.maximum(s - 2 * (last + 1), 0), 0)),
        scratch_shapes=[
            pltpu.VMEM((B, H0), jnp.float32),
            pltpu.VMEM((B, H1), jnp.float32),
            pltpu.VMEM((2, H0), jnp.float32),
            pltpu.VMEM((2, H1), jnp.float32),
        ],
    )

    return pl.pallas_call(
        _fused_kernel,
        out_shape=jax.ShapeDtypeStruct((B, output_dim), jnp.float32),
        grid_spec=grid_spec,
        compiler_params=pltpu.CompilerParams(
            dimension_semantics=("arbitrary",),
            vmem_limit_bytes=48 * 1024 * 1024),
        cost_estimate=pl.CostEstimate(
            flops=flops,
            transcendentals=B * output_dim + H0 + H1,
            bytes_accessed=bytes_accessed),
    )(latent_vec, class_onehot, w1, bn1, w2, bn2, w3a_pad, w3b_pad, b3_pad)
